# Initial kernel scaffold; baseline (speedup 1.0000x reference)
import jax
import jax.numpy as jnp
from jax.experimental import pallas as pl


def kernel(x, edge_index, edge_attr, batch, params):
    raise NotImplementedError("write your pallas kernel here")


# speedup vs baseline: 5.2615x; 5.2615x over previous
import functools

import jax
import jax.numpy as jnp
from jax import lax
from jax.experimental import pallas as pl
from jax.experimental.pallas import tpu as pltpu
from jax.experimental.pallas import tpu_sc as plsc

N = 10000
E = 160000
EMB = 256
HALF = 128
FEAT = 512
NUM_LAYER = 5
NUM_GRAPHS = 256
EPS = 1e-5

NSUB = 16
NCORE = 2
CH = 128
NPAD = 10240
NCHUNK_N = 5
ECHUNKS = 79
EPADTOT = NSUB * ECHUNKS * CH
ROWS_PER_SUB = NPAD // NSUB
BN = 1000
NBLK = N // BN

f32 = jnp.float32
i32 = jnp.int32

_SC_MESH = dict(core_axis_name="c", subcore_axis_name="s",
                num_cores=NCORE, num_subcores=NSUB)


def _zero_rows(ref, nrows, width):
    zero16 = jnp.zeros((16,), f32)

    def zrow(r, _):
        for k in range(width // 16):
            ref[r, pl.ds(k * 16, 16)] = zero16
        return 0

    lax.fori_loop(0, nrows, zrow, 0)


def _sc_init_body(comb0, comb1, xi_hbm, oh_hbm, dst_hbm,
                  h0_out, h1_out, c_out,
                  xi_v, dst_v, rows_v, c_sh, sem):
    c = lax.axis_index("c")
    s = lax.axis_index("s")
    pltpu.sync_copy(xi_hbm.at[s], xi_v)

    def gather_phase(tbl, hout):
        for j in range(NCHUNK_N):
            pltpu.sync_copy(tbl.at[xi_v.at[j]], rows_v)
            pltpu.sync_copy(rows_v, hout.at[pl.ds((s * NCHUNK_N + j) * CH, CH)])

    pl.when(c == 0)(lambda: gather_phase(comb0, h0_out))
    pl.when(c == 1)(lambda: gather_phase(comb1, h1_out))

    @pl.when(c == 0)
    def _():
        pltpu.sync_copy(dst_hbm.at[s], dst_v)
        _zero_rows(rows_v, CH, HALF)
        for k in range(NCHUNK_N):
            pltpu.sync_copy(rows_v, c_sh.at[pl.ds(s * ROWS_PER_SUB + k * CH, CH)])
        plsc.subcore_barrier()

        def step(j, _):
            pltpu.sync_copy(oh_hbm.at[s, j], rows_v)
            pltpu.sync_copy(rows_v, c_sh.at[dst_v.at[j]], add=True)
            return 0

        lax.fori_loop(0, ECHUNKS, step, 0)
        plsc.subcore_barrier()
        pltpu.sync_copy(c_sh.at[pl.ds(s * ROWS_PER_SUB, ROWS_PER_SUB)],
                        c_out.at[pl.ds(s * ROWS_PER_SUB, ROWS_PER_SUB)])


def _sc_init(comb0, comb1, xi, oh, dstp):
    kfn = pl.kernel(
        _sc_init_body,
        out_type=(
            jax.ShapeDtypeStruct((NPAD, HALF), f32),
            jax.ShapeDtypeStruct((NPAD, HALF), f32),
            jax.ShapeDtypeStruct((NPAD, HALF), f32),
        ),
        mesh=plsc.VectorSubcoreMesh(**_SC_MESH),
        scratch_types=(
            pltpu.VMEM((NCHUNK_N, CH), i32),
            pltpu.VMEM((ECHUNKS, CH), i32),
            pltpu.VMEM((CH, HALF), f32),
            pltpu.VMEM_SHARED((NPAD, HALF), f32),
            pltpu.SemaphoreType.DMA,
        ),
    )
    return kfn(comb0, comb1, xi, oh, dstp)


def _sc_layer_body(h0, h1, src_hbm, dst_hbm, s0_out, s1_out,
                   src_v, dst_v, rows_v, s_sh, sem):
    c = lax.axis_index("c")
    s = lax.axis_index("s")
    pltpu.sync_copy(src_hbm.at[s], src_v)
    pltpu.sync_copy(dst_hbm.at[s], dst_v)
    _zero_rows(rows_v, CH, HALF)
    for k in range(NCHUNK_N):
        pltpu.sync_copy(rows_v, s_sh.at[pl.ds(s * ROWS_PER_SUB + k * CH, CH)])
    plsc.subcore_barrier()

    def run(h_ref):
        def step(j, _):
            pltpu.sync_copy(h_ref.at[src_v.at[j]], rows_v)
            pltpu.sync_copy(rows_v, s_sh.at[dst_v.at[j]], add=True)
            return 0

        lax.fori_loop(0, ECHUNKS, step, 0)

    pl.when(c == 0)(lambda: run(h0))
    pl.when(c == 1)(lambda: run(h1))
    plsc.subcore_barrier()

    def out(s_ref):
        pltpu.sync_copy(s_sh.at[pl.ds(s * ROWS_PER_SUB, ROWS_PER_SUB)],
                        s_ref.at[pl.ds(s * ROWS_PER_SUB, ROWS_PER_SUB)])

    pl.when(c == 0)(lambda: out(s0_out))
    pl.when(c == 1)(lambda: out(s1_out))


def _sc_layer(h0, h1, srcp, dstp):
    kfn = pl.kernel(
        _sc_layer_body,
        out_type=(
            jax.ShapeDtypeStruct((NPAD, HALF), f32),
            jax.ShapeDtypeStruct((NPAD, HALF), f32),
        ),
        mesh=plsc.VectorSubcoreMesh(**_SC_MESH),
        scratch_types=(
            pltpu.VMEM((ECHUNKS, CH), i32),
            pltpu.VMEM((ECHUNKS, CH), i32),
            pltpu.VMEM((CH, HALF), f32),
            pltpu.VMEM_SHARED((NPAD, HALF), f32),
            pltpu.SemaphoreType.DMA,
        ),
    )
    return kfn(h0, h1, srcp, dstp)


def _k1_body(s0, s1, h0, h1, cc, m, be, w1, b1, w2, b2,
             z_out, sums):
    i = pl.program_id(0)
    hp = lax.Precision.HIGHEST
    agg = jnp.concatenate([s0[...] + h0[...], s1[...] + h1[...]], axis=1)
    agg += jnp.dot(cc[...], m[...], precision=hp, preferred_element_type=f32)
    agg += be[...]
    z1 = jnp.maximum(jnp.dot(agg, w1[...], preferred_element_type=f32) + b1[...], 0.0)
    z2 = jnp.dot(z1, w2[...], preferred_element_type=f32) + b2[...]
    z_out[...] = z2

    @pl.when(i == 0)
    def _():
        sums[...] = jnp.zeros_like(sums)

    sums[...] += jnp.sum(z2, axis=0, keepdims=True)


def _tc_mlp(s0, s1, h0, h1, cc, m, be, w1, b1, w2, b2):
    blk = lambda r, cdim: pl.BlockSpec((r, cdim), lambda i: (i, 0))
    full = lambda a, b: pl.BlockSpec((a, b), lambda i: (0, 0))
    return pl.pallas_call(
        _k1_body,
        grid=(NBLK,),
        in_specs=[
            blk(BN, HALF), blk(BN, HALF), blk(BN, HALF), blk(BN, HALF),
            blk(BN, HALF),
            full(HALF, EMB), full(1, EMB), full(EMB, FEAT), full(1, FEAT),
            full(FEAT, EMB), full(1, EMB),
        ],
        out_specs=[
            pl.BlockSpec((BN, EMB), lambda i: (i, 0)),
            full(1, EMB),
        ],
        out_shape=[
            jax.ShapeDtypeStruct((N, EMB), f32),
            jax.ShapeDtypeStruct((1, EMB), f32),
        ],
    )(s0, s1, h0, h1, cc, m, be, w1, b1, w2, b2)


def _k2_body(z, sums, gamma, beta, h0_out, h1_out, ssq, *, relu):
    i = pl.program_id(0)
    mean = sums[...] / N

    @pl.when(i == 0)
    def _():
        ssq[...] = jnp.zeros_like(ssq)

    @pl.when(i < NBLK)
    def _():
        d = z[...] - mean
        ssq[...] += jnp.sum(d * d, axis=0, keepdims=True)

    @pl.when(i >= NBLK)
    def _():
        var = ssq[...] / N
        hn = (z[...] - mean) / jnp.sqrt(var + EPS) * gamma[...] + beta[...]
        if relu:
            hn = jnp.maximum(hn, 0.0)
        h0_out[...] = hn[:, :HALF]
        h1_out[...] = hn[:, HALF:]


def _tc_bn(z, sums, gamma, beta, relu):
    full = lambda a, b: pl.BlockSpec((a, b), lambda i: (0, 0))
    return pl.pallas_call(
        functools.partial(_k2_body, relu=relu),
        grid=(2 * NBLK,),
        in_specs=[
            pl.BlockSpec((BN, EMB), lambda i: (i % NBLK, 0)),
            full(1, EMB), full(1, EMB), full(1, EMB),
        ],
        out_specs=[
            pl.BlockSpec((BN, HALF), lambda i: (i % NBLK, 0)),
            pl.BlockSpec((BN, HALF), lambda i: (i % NBLK, 0)),
        ],
        out_shape=[
            jax.ShapeDtypeStruct((NPAD, HALF), f32),
            jax.ShapeDtypeStruct((NPAD, HALF), f32),
        ],
        scratch_shapes=[pltpu.VMEM((1, EMB), f32)],
    )(z, sums, gamma, beta)


def _softplus(x):
    return jnp.maximum(x, 0.0) + jnp.log1p(jnp.exp(-jnp.abs(x)))


def _kf_body(h0, h1, bat, fw, fb, w1, b1, w2, b2, w3p, b3p,
             hg_out, pred_out, acc0, acc1, cnt):
    i = pl.program_id(0)
    bcol = bat[0]
    iota = lax.broadcasted_iota(i32, (BN, NUM_GRAPHS), 1)
    ohf = (bcol == iota).astype(f32)

    @pl.when(i == 0)
    def _():
        acc0[...] = jnp.zeros_like(acc0)
        acc1[...] = jnp.zeros_like(acc1)
        cnt[...] = jnp.zeros_like(cnt)

    dn = (((0,), (0,)), ((), ()))
    hp = lax.Precision.HIGHEST
    acc0[...] += lax.dot_general(ohf, h0[...], dn, precision=hp,
                                 preferred_element_type=f32)
    acc1[...] += lax.dot_general(ohf, h1[...], dn, precision=hp,
                                 preferred_element_type=f32)
    cnt[...] += lax.dot_general(ohf, jnp.ones((BN, HALF), f32), dn, precision=hp,
                                preferred_element_type=f32)

    @pl.when(i == NBLK - 1)
    def _():
        cmax = jnp.maximum(cnt[...][:, 0:1], 1.0)
        pool = jnp.concatenate([acc0[...], acc1[...]], axis=1) / cmax
        hg = jnp.dot(pool, fw[...], preferred_element_type=f32) + fb[...]
        hg_out[...] = hg
        p = _softplus(jnp.dot(hg, w1[...], preferred_element_type=f32) + b1[...])
        p = _softplus(jnp.dot(p, w2[...], preferred_element_type=f32) + b2[...])
        pred_out[...] = jnp.dot(p, w3p[...], preferred_element_type=f32) + b3p[...]


def _tc_pool_head(h0, h1, bat3, fw, fb, w1, b1, w2, b2, w3p, b3p):
    full = lambda a, b: pl.BlockSpec((a, b), lambda i: (0, 0))
    return pl.pallas_call(
        _kf_body,
        grid=(NBLK,),
        in_specs=[
            pl.BlockSpec((BN, HALF), lambda i: (i, 0)),
            pl.BlockSpec((BN, HALF), lambda i: (i, 0)),
            pl.BlockSpec((1, BN, 1), lambda i: (i, 0, 0)),
            full(EMB, FEAT), full(1, FEAT),
            full(FEAT, FEAT // 2), full(1, FEAT // 2),
            full(FEAT // 2, FEAT // 2), full(1, FEAT // 2),
            full(FEAT // 2, HALF), full(1, HALF),
        ],
        out_specs=[
            full(NUM_GRAPHS, FEAT),
            full(NUM_GRAPHS, HALF),
        ],
        out_shape=[
            jax.ShapeDtypeStruct((NUM_GRAPHS, FEAT), f32),
            jax.ShapeDtypeStruct((NUM_GRAPHS, HALF), f32),
        ],
        scratch_shapes=[
            pltpu.VMEM((NUM_GRAPHS, HALF), f32),
            pltpu.VMEM((NUM_GRAPHS, HALF), f32),
            pltpu.VMEM((NUM_GRAPHS, HALF), f32),
        ],
    )(h0, h1, bat3, fw, fb, w1, b1, w2, b2, w3p, b3p)


def kernel(x, edge_index, edge_attr, batch, params):
    comb = (params['x_emb1'][:, None, :] + params['x_emb2'][None, :, :])
    comb = comb.reshape(-1, EMB)
    comb0 = comb[:, :HALF]
    comb1 = comb[:, HALF:]
    nct = params['x_emb2'].shape[0]
    xi = (x[:, 0] * nct + x[:, 1]).astype(i32)
    xi = jnp.concatenate([xi, jnp.zeros((NPAD - N,), i32)])
    xi = xi.reshape(NSUB, NCHUNK_N, CH)

    pad_e = EPADTOT - E
    src = jnp.concatenate([edge_index[0].astype(i32),
                           jnp.zeros((pad_e,), i32)]).reshape(NSUB, ECHUNKS, CH)
    dst = jnp.concatenate([edge_index[1].astype(i32),
                           jnp.full((pad_e,), NPAD - 1, i32)]).reshape(NSUB, ECHUNKS, CH)

    cols = jnp.arange(HALF, dtype=edge_attr.dtype)
    oh = ((edge_attr[:, 0:1] == cols[None, :]) |
          (edge_attr[:, 1:2] + 8 == cols[None, :])).astype(f32)
    oh = jnp.concatenate([oh, jnp.zeros((pad_e, HALF), f32)])
    oh = oh.reshape(NSUB, ECHUNKS, CH, HALF)

    bat3 = batch.astype(i32).reshape(NBLK, BN, 1)

    h0, h1, cc = _sc_init(comb0, comb1, xi, oh, dst)

    for l in range(NUM_LAYER):
        e1 = params['edge_emb1'][l]
        e2 = params['edge_emb2'][l]
        m = jnp.zeros((HALF, EMB), f32).at[0:5].set(e1).at[8:11].set(e2)
        b_edge = (e1[4] + e2[0]).reshape(1, EMB)
        w1 = params['w1'][l]
        b1 = params['b1'][l].reshape(1, -1)
        w2 = params['w2'][l]
        b2 = params['b2'][l].reshape(1, -1)
        gamma = params['bn_gamma'][l].reshape(1, -1)
        beta = params['bn_beta'][l].reshape(1, -1)

        s0, s1 = _sc_layer(h0, h1, src, dst)
        z, sums = _tc_mlp(s0, s1, h0, h1, cc, m, b_edge, w1, b1, w2, b2)
        h0, h1 = _tc_bn(z, sums, gamma, beta, relu=(l != NUM_LAYER - 1))

    fw = params['feat_w']
    w3p = jnp.pad(params['head_w3'], ((0, 0), (0, HALF - 1)))
    b3p = jnp.pad(params['head_b3'].reshape(1, 1), ((0, 0), (0, HALF - 1)))
    hg, pred_full = _tc_pool_head(
        h0, h1, bat3,
        fw, params['feat_b'].reshape(1, -1),
        params['head_w1'], params['head_b1'].reshape(1, -1),
        params['head_w2'], params['head_b2'].reshape(1, -1),
        w3p, b3p)
    return (hg, pred_full[:, :1])
